# baseline (speedup 1.0000x reference)
import jax
import jax.numpy as jnp
from jax.experimental import pallas as pl

_NUM_EXPERTS = 16
_BLOCK_T = 1024


def _routing_kernel(x_ref, w_ref, b_ref, rw_ref, idx_ref, probs_ref, top2p_ref):
    x = x_ref[...]
    w = w_ref[...]
    logits = jax.lax.dot_general(
        x, w, (((1,), (1,)), ((), ())), preferred_element_type=jnp.float32
    ) + b_ref[...]

    m = jnp.max(logits, axis=-1, keepdims=True)
    e = jnp.exp(logits - m)
    denom = jnp.sum(e, axis=-1, keepdims=True)
    probs = e / denom
    p1 = 1.0 / denom

    lane = jax.lax.broadcasted_iota(jnp.int32, probs.shape, 1)
    i1 = jnp.min(jnp.where(probs == p1, lane, _NUM_EXPERTS), axis=-1, keepdims=True)
    masked = jnp.where(lane == i1, -jnp.inf, probs)
    p2 = jnp.max(masked, axis=-1, keepdims=True)
    i2 = jnp.min(jnp.where(masked == p2, lane, _NUM_EXPERTS), axis=-1, keepdims=True)

    s = p1 + p2
    p1n = p1 / s
    p2n = p2 / s

    rw_ref[...] = jnp.where(lane == i1, p1n, jnp.where(lane == i2, p2n, 0.0))
    probs_ref[...] = probs
    idx_ref[...] = jnp.concatenate([i1, i2], axis=-1)
    top2p_ref[...] = jnp.concatenate([p1n, p2n], axis=-1)


@jax.jit
def kernel(hidden_states, W, b):
    batch_size, seq_len, hidden_dim = hidden_states.shape
    n_tokens = batch_size * seq_len
    x = hidden_states.reshape(n_tokens, hidden_dim)
    b2 = b.reshape(1, _NUM_EXPERTS)

    grid = (n_tokens // _BLOCK_T,)
    out = pl.pallas_call(
        _routing_kernel,
        grid=grid,
        in_specs=[
            pl.BlockSpec((_BLOCK_T, hidden_dim), lambda i: (i, 0)),
            pl.BlockSpec((_NUM_EXPERTS, hidden_dim), lambda i: (0, 0)),
            pl.BlockSpec((1, _NUM_EXPERTS), lambda i: (0, 0)),
        ],
        out_specs=[
            pl.BlockSpec((_BLOCK_T, _NUM_EXPERTS), lambda i: (i, 0)),
            pl.BlockSpec((_BLOCK_T, 2), lambda i: (i, 0)),
            pl.BlockSpec((_BLOCK_T, _NUM_EXPERTS), lambda i: (i, 0)),
            pl.BlockSpec((_BLOCK_T, 2), lambda i: (i, 0)),
        ],
        out_shape=[
            jax.ShapeDtypeStruct((n_tokens, _NUM_EXPERTS), jnp.float32),
            jax.ShapeDtypeStruct((n_tokens, 2), jnp.int32),
            jax.ShapeDtypeStruct((n_tokens, _NUM_EXPERTS), jnp.float32),
            jax.ShapeDtypeStruct((n_tokens, 2), jnp.float32),
        ],
    )(x, W, b2)
    routing_weights, top2_indices, router_probs, top2_probs = out
    return (routing_weights, top2_indices, router_probs, top2_probs)


# speedup vs baseline: 1.0633x; 1.0633x over previous
import jax
import jax.numpy as jnp
from jax.experimental import pallas as pl

_NUM_EXPERTS = 16
_BLOCK_T = 4096


def _routing_kernel(x_ref, w_ref, b_ref, rw_ref, idx_ref, probs_ref, top2p_ref):
    x = x_ref[...]
    w = w_ref[...]
    logits = jax.lax.dot_general(
        x, w, (((1,), (1,)), ((), ())), preferred_element_type=jnp.float32
    ) + b_ref[...]

    m = jnp.max(logits, axis=-1, keepdims=True)
    e = jnp.exp(logits - m)
    denom = jnp.sum(e, axis=-1, keepdims=True)
    probs = e / denom
    p1 = 1.0 / denom

    lane = jax.lax.broadcasted_iota(jnp.int32, probs.shape, 1)
    i1 = jnp.min(jnp.where(probs == p1, lane, _NUM_EXPERTS), axis=-1, keepdims=True)
    masked = jnp.where(lane == i1, -jnp.inf, probs)
    p2 = jnp.max(masked, axis=-1, keepdims=True)
    i2 = jnp.min(jnp.where(masked == p2, lane, _NUM_EXPERTS), axis=-1, keepdims=True)

    s = p1 + p2
    p1n = p1 / s
    p2n = p2 / s

    rw_ref[...] = jnp.where(lane == i1, p1n, jnp.where(lane == i2, p2n, 0.0))
    probs_ref[...] = probs
    idx_ref[...] = jnp.concatenate([i1, i2], axis=-1)
    top2p_ref[...] = jnp.concatenate([p1n, p2n], axis=-1)


@jax.jit
def kernel(hidden_states, W, b):
    batch_size, seq_len, hidden_dim = hidden_states.shape
    n_tokens = batch_size * seq_len
    x = hidden_states.reshape(n_tokens, hidden_dim)
    b2 = b.reshape(1, _NUM_EXPERTS)

    grid = (n_tokens // _BLOCK_T,)
    out = pl.pallas_call(
        _routing_kernel,
        grid=grid,
        in_specs=[
            pl.BlockSpec((_BLOCK_T, hidden_dim), lambda i: (i, 0)),
            pl.BlockSpec((_NUM_EXPERTS, hidden_dim), lambda i: (0, 0)),
            pl.BlockSpec((1, _NUM_EXPERTS), lambda i: (0, 0)),
        ],
        out_specs=[
            pl.BlockSpec((_BLOCK_T, _NUM_EXPERTS), lambda i: (i, 0)),
            pl.BlockSpec((_BLOCK_T, 2), lambda i: (i, 0)),
            pl.BlockSpec((_BLOCK_T, _NUM_EXPERTS), lambda i: (i, 0)),
            pl.BlockSpec((_BLOCK_T, 2), lambda i: (i, 0)),
        ],
        out_shape=[
            jax.ShapeDtypeStruct((n_tokens, _NUM_EXPERTS), jnp.float32),
            jax.ShapeDtypeStruct((n_tokens, 2), jnp.int32),
            jax.ShapeDtypeStruct((n_tokens, _NUM_EXPERTS), jnp.float32),
            jax.ShapeDtypeStruct((n_tokens, 2), jnp.float32),
        ],
    )(x, W, b2)
    routing_weights, top2_indices, router_probs, top2_probs = out
    return (routing_weights, top2_indices, router_probs, top2_probs)
